# baseline (speedup 1.0000x reference)
import jax
import jax.numpy as jnp
from jax import lax
from jax.experimental import pallas as pl
from jax.experimental.pallas import tpu as pltpu
from jax.experimental.pallas import tpu_sc as plsc

DEPTH = 1000
BATCH = 16384
NC = 2
NS = 16
L = 16
NW = NC * NS
BPW = BATCH // NW
CHUNK = 64
NCH = BPW // CHUNK
NG = CHUNK // L


def _body(x_hbm, out_hbm, idx_v, buf0, buf1, sem0, sem1):
    wid = lax.axis_index("s") * NC + lax.axis_index("c")
    base = wid * BPW
    pltpu.sync_copy(x_hbm.at[pl.ds(base, BPW)], idx_v)

    zeros = jnp.zeros((L,), jnp.float32)
    ones = jnp.full((L,), 1.0, jnp.float32)
    lane = lax.iota(jnp.int32, L)

    def zfill(i, carry):
        buf0[pl.ds(i * L, L)] = zeros
        buf1[pl.ds(i * L, L)] = zeros
        return carry

    lax.fori_loop(0, CHUNK * DEPTH // L, zfill, 0)

    def scatter(buf, ch, val):
        for g in range(NG):
            col = idx_v[pl.ds(ch * CHUNK + g * L, L)]
            flat = (lane + (g * L)) * DEPTH + col
            plsc.store_scatter(buf, [flat], val)

    bufs = (buf0, buf1)
    sems = (sem0, sem1)
    copies = {}
    for ch in range(NCH):
        b = ch % 2
        buf = bufs[b]
        if ch >= 2:
            copies[ch - 2].wait()
            scatter(buf, ch - 2, zeros)
        scatter(buf, ch, ones)
        row0 = base + ch * CHUNK
        cp = pltpu.make_async_copy(
            buf, out_hbm.at[pl.ds(row0 * DEPTH, CHUNK * DEPTH)], sems[b])
        cp.start()
        copies[ch] = cp
    copies[NCH - 2].wait()
    copies[NCH - 1].wait()


@jax.jit
def _onehot(x):
    mesh = plsc.VectorSubcoreMesh(core_axis_name="c", subcore_axis_name="s")
    k = pl.kernel(
        _body,
        out_type=jax.ShapeDtypeStruct((BATCH * DEPTH,), jnp.float32),
        mesh=mesh,
        scratch_types=[
            pltpu.VMEM((BPW,), jnp.int32),
            pltpu.VMEM((CHUNK * DEPTH,), jnp.float32),
            pltpu.VMEM((CHUNK * DEPTH,), jnp.float32),
            pltpu.SemaphoreType.DMA,
            pltpu.SemaphoreType.DMA,
        ],
        compiler_params=pltpu.CompilerParams(needs_layout_passes=False),
    )
    return k(x)


def kernel(X_in, ones):
    out = _onehot(X_in.astype(jnp.int32))
    return out.reshape(BATCH, DEPTH)


# speedup vs baseline: 1.7680x; 1.7680x over previous
import jax
import jax.numpy as jnp
from jax import lax
from jax.experimental import pallas as pl
from jax.experimental.pallas import tpu as pltpu
from jax.experimental.pallas import tpu_sc as plsc

DEPTH = 1000
BATCH = 16384
NC = 2
NS = 16
L = 16
NW = NC * NS
BPW = BATCH // NW
CHUNK = 32
NCH = BPW // CHUNK
NG = CHUNK // L


def _body(x_hbm, out_hbm, idx_v, buf0, buf1, sem0, sem1):
    wid = lax.axis_index("s") * NC + lax.axis_index("c")
    base = wid * BPW
    pltpu.sync_copy(x_hbm.at[pl.ds(base, BPW)], idx_v)

    zeros = jnp.zeros((L,), jnp.float32)
    ones = jnp.full((L,), 1.0, jnp.float32)
    lane = lax.iota(jnp.int32, L)

    def zfill(r, carry):
        for j in range(DEPTH // L):
            buf0[r, pl.ds(j * L, L)] = zeros
            buf1[r, pl.ds(j * L, L)] = zeros
        buf0[r, pl.ds(DEPTH - L, L)] = zeros
        buf1[r, pl.ds(DEPTH - L, L)] = zeros
        return carry

    lax.fori_loop(0, CHUNK, zfill, 0)

    def scatter(buf, ch, val):
        for g in range(NG):
            col = idx_v[pl.ds(ch * CHUNK + g * L, L)]
            row = lane + (g * L)
            plsc.store_scatter(buf, [row, col], val)

    bufs = (buf0, buf1)
    sems = (sem0, sem1)
    copies = {}
    for ch in range(NCH):
        b = ch % 2
        buf = bufs[b]
        if ch >= 2:
            copies[ch - 2].wait()
            scatter(buf, ch - 2, zeros)
        scatter(buf, ch, ones)
        row0 = base + ch * CHUNK
        cp = pltpu.make_async_copy(
            buf, out_hbm.at[pl.ds(row0, CHUNK)], sems[b])
        cp.start()
        copies[ch] = cp
    copies[NCH - 2].wait()
    copies[NCH - 1].wait()


@jax.jit
def _onehot(x):
    mesh = plsc.VectorSubcoreMesh(core_axis_name="c", subcore_axis_name="s")
    k = pl.kernel(
        _body,
        out_type=jax.ShapeDtypeStruct((BATCH, DEPTH), jnp.float32),
        mesh=mesh,
        scratch_types=[
            pltpu.VMEM((BPW,), jnp.int32),
            pltpu.VMEM((CHUNK, DEPTH), jnp.float32),
            pltpu.VMEM((CHUNK, DEPTH), jnp.float32),
            pltpu.SemaphoreType.DMA,
            pltpu.SemaphoreType.DMA,
        ],
        compiler_params=pltpu.CompilerParams(
            needs_layout_passes=False,
            use_tc_tiling_on_sc=True,
        ),
    )
    return k(x)


def kernel(X_in, ones):
    return _onehot(X_in.astype(jnp.int32))


# speedup vs baseline: 3.9534x; 2.2361x over previous
import jax
import jax.numpy as jnp
from jax import lax
from jax.experimental import pallas as pl
from jax.experimental.pallas import tpu as pltpu
from jax.experimental.pallas import tpu_sc as plsc

DEPTH = 1000
BATCH = 16384
NC = 2
NS = 16
L = 16
NW = NC * NS
SPW = BATCH // NW
CHUNK = 128
NCH = SPW // CHUNK
NG = CHUNK // L


def _body(x_hbm, out_hbm, idx_v, buf, sem):
    wid = lax.axis_index("s") * NC + lax.axis_index("c")
    base = wid * SPW
    pltpu.sync_copy(x_hbm.at[pl.ds(base, SPW)], idx_v)

    zeros = jnp.zeros((L,), jnp.float32)
    ones = jnp.full((L,), 1.0, jnp.float32)
    lane = lax.iota(jnp.int32, L)

    def zfill(r, carry):
        for j in range(CHUNK // L):
            buf[r, pl.ds(j * L, L)] = zeros
        return carry

    lax.fori_loop(0, DEPTH, zfill, 0)

    def scatter(ch, val):
        for g in range(NG):
            row = idx_v[pl.ds(ch * CHUNK + g * L, L)]
            col = lane + (g * L)
            plsc.store_scatter(buf, [row, col], val)

    for ch in range(NCH):
        scatter(ch, ones)
        col0 = base + ch * CHUNK
        cp = pltpu.make_async_copy(
            buf, out_hbm.at[:, pl.ds(col0, CHUNK)], sem)
        cp.start()
        cp.wait()
        if ch < NCH - 1:
            scatter(ch, zeros)


@jax.jit
def _onehot_t(x):
    mesh = plsc.VectorSubcoreMesh(core_axis_name="c", subcore_axis_name="s")
    k = pl.kernel(
        _body,
        out_type=jax.ShapeDtypeStruct((DEPTH, BATCH), jnp.float32),
        mesh=mesh,
        scratch_types=[
            pltpu.VMEM((SPW,), jnp.int32),
            pltpu.VMEM((DEPTH, CHUNK), jnp.float32),
            pltpu.SemaphoreType.DMA,
        ],
        compiler_params=pltpu.CompilerParams(
            needs_layout_passes=False,
            use_tc_tiling_on_sc=True,
        ),
    )
    return k(x)


def kernel(X_in, ones):
    return _onehot_t(X_in.astype(jnp.int32)).T
